# Initial kernel scaffold; baseline (speedup 1.0000x reference)
import jax
import jax.numpy as jnp
from jax.experimental import pallas as pl


def kernel(hashes, weights, table):
    raise NotImplementedError("write your pallas kernel here")


# speedup vs baseline: 2.6785x; 2.6785x over previous
import functools

import jax
import jax.numpy as jnp
from jax import lax
from jax.experimental import pallas as pl
from jax.experimental.pallas import tpu as pltpu
from jax.experimental.pallas import tpu_sc as plsc

_B = 16384
_L = 50
_D = 32
_LANES = 16

_C = 32
_E = _C * _L
_LPAD = 64
_G = 100
_NG = _E // _G


def _bag_kernel(h_hbm, w_hbm, t_hbm, o_hbm, idx_v, wv, rows_v, out_v, sem,
                *, num_cores, num_chunks):
    wid = lax.axis_index("s") * num_cores + lax.axis_index("c")

    def chunk(ci, carry):
        pltpu.sync_copy(h_hbm.at[wid, ci], idx_v)
        pltpu.sync_copy(w_hbm.at[wid, ci], wv)
        cps = [
            pltpu.async_copy(t_hbm.at[idx_v.at[j]],
                             rows_v.at[pl.ds(j * _G, _G)], sem)
            for j in range(_NG)
        ]
        for cp in cps:
            cp.wait()

        def bag(b, carry2):
            base = b * _L
            wregs = [wv[b, pl.ds(k * _LANES, _LANES)] for k in range(4)]
            a0 = jnp.zeros((_LANES,), jnp.float32)
            a1 = jnp.zeros((_LANES,), jnp.float32)
            for e in range(_L):
                w = wregs[e // _LANES][e % _LANES]
                a0 = a0 + w * rows_v[base + e, pl.ds(0, _LANES)]
                a1 = a1 + w * rows_v[base + e, pl.ds(_LANES, _LANES)]
            out_v[b, pl.ds(0, _LANES)] = a0
            out_v[b, pl.ds(_LANES, _LANES)] = a1
            return carry2

        lax.fori_loop(0, _C, bag, 0)
        pltpu.sync_copy(out_v, o_hbm.at[pl.ds((wid * num_chunks + ci) * _C, _C)])
        return carry

    lax.fori_loop(0, num_chunks, chunk, 0)


def kernel(hashes, weights, table):
    info = plsc.get_sparse_core_info()
    nw = info.num_cores * info.num_subcores
    bags_per_worker = _B // nw
    num_chunks = bags_per_worker // _C

    h4 = hashes.reshape(nw, num_chunks, _NG, _G)
    wpad = jnp.pad(weights, ((0, 0), (0, _LPAD - _L)))
    w4 = wpad.reshape(nw, num_chunks, _C, _LPAD)

    mesh = plsc.VectorSubcoreMesh(core_axis_name="c", subcore_axis_name="s")
    run = functools.partial(
        pl.kernel,
        mesh=mesh,
        compiler_params=pltpu.CompilerParams(use_tc_tiling_on_sc=False),
        out_type=jax.ShapeDtypeStruct((_B, _D), jnp.float32),
        scratch_types=[
            pltpu.VMEM((_NG, _G), jnp.int32),
            pltpu.VMEM((_C, _LPAD), jnp.float32),
            pltpu.VMEM((_E, _D), jnp.float32),
            pltpu.VMEM((_C, _D), jnp.float32),
            pltpu.SemaphoreType.DMA,
        ],
    )(functools.partial(_bag_kernel, num_cores=info.num_cores,
                        num_chunks=num_chunks))
    return run(h4, w4, table)


# speedup vs baseline: 2.6992x; 1.0077x over previous
import functools

import jax
import jax.numpy as jnp
from jax import lax
from jax.experimental import pallas as pl
from jax.experimental.pallas import tpu as pltpu
from jax.experimental.pallas import tpu_sc as plsc

_B = 16384
_L = 50
_D = 32
_LANES = 16

_C = 32
_E = _C * _L
_GB = 8
_GE = _GB * _L
_G = 100
_NG = _E // _G


def _bag_kernel(h_hbm, w_hbm, t_hbm, o_hbm, idx_v, wv, rows_v, out_v, sem,
                *, num_cores, num_chunks):
    wid = lax.axis_index("s") * num_cores + lax.axis_index("c")

    def chunk(ci, carry):
        pltpu.sync_copy(h_hbm.at[wid, ci], idx_v)
        pltpu.sync_copy(w_hbm.at[wid, ci], wv)
        cps = [
            pltpu.async_copy(t_hbm.at[idx_v.at[j]],
                             rows_v.at[pl.ds(j * _G, _G)], sem)
            for j in range(_NG)
        ]
        for cp in cps:
            cp.wait()

        def group(g, carry2):
            wregs = [wv[g, pl.ds(k * _LANES, _LANES)] for k in range(_GE // _LANES)]
            rbase = g * _GE
            for j in range(_GB):
                a0 = jnp.zeros((_LANES,), jnp.float32)
                a1 = jnp.zeros((_LANES,), jnp.float32)
                for e in range(_L):
                    f = j * _L + e
                    w = wregs[f // _LANES][f % _LANES]
                    a0 = a0 + w * rows_v[rbase + f, pl.ds(0, _LANES)]
                    a1 = a1 + w * rows_v[rbase + f, pl.ds(_LANES, _LANES)]
                b = g * _GB + j
                out_v[b, pl.ds(0, _LANES)] = a0
                out_v[b, pl.ds(_LANES, _LANES)] = a1
            return carry2

        lax.fori_loop(0, _C // _GB, group, 0)
        pltpu.sync_copy(out_v, o_hbm.at[pl.ds((wid * num_chunks + ci) * _C, _C)])
        return carry

    lax.fori_loop(0, num_chunks, chunk, 0)


def kernel(hashes, weights, table):
    info = plsc.get_sparse_core_info()
    nw = info.num_cores * info.num_subcores
    bags_per_worker = _B // nw
    num_chunks = bags_per_worker // _C

    h4 = hashes.reshape(nw, num_chunks, _NG, _G)
    w4 = weights.reshape(nw, num_chunks, _C // _GB, _GE)

    mesh = plsc.VectorSubcoreMesh(core_axis_name="c", subcore_axis_name="s")
    run = functools.partial(
        pl.kernel,
        mesh=mesh,
        compiler_params=pltpu.CompilerParams(use_tc_tiling_on_sc=False),
        out_type=jax.ShapeDtypeStruct((_B, _D), jnp.float32),
        scratch_types=[
            pltpu.VMEM((_NG, _G), jnp.int32),
            pltpu.VMEM((_C // _GB, _GE), jnp.float32),
            pltpu.VMEM((_E, _D), jnp.float32),
            pltpu.VMEM((_C, _D), jnp.float32),
            pltpu.SemaphoreType.DMA,
        ],
    )(functools.partial(_bag_kernel, num_cores=info.num_cores,
                        num_chunks=num_chunks))
    return run(h4, w4, table)


# speedup vs baseline: 2.7616x; 1.0231x over previous
import functools

import jax
import jax.numpy as jnp
from jax import lax
from jax.experimental import pallas as pl
from jax.experimental.pallas import tpu as pltpu
from jax.experimental.pallas import tpu_sc as plsc

_B = 16384
_L = 50
_D = 32
_LANES = 16

_C = 64
_E = _C * _L
_G = 128
_NG = _E // _G
_GB = 8
_GE = _GB * _L


def _bag_kernel(h_hbm, w_hbm, t_hbm, o_hbm, idx_v, wv, rows_v, out_v, sem,
                *, num_cores, num_chunks):
    wid = lax.axis_index("s") * num_cores + lax.axis_index("c")

    def chunk(ci, carry):
        cid = wid * num_chunks + ci
        pltpu.sync_copy(h_hbm.at[pl.ds(cid * _NG, _NG)], idx_v)
        pltpu.sync_copy(w_hbm.at[pl.ds(cid * _E, _E)], wv)
        cps = [
            pltpu.async_copy(t_hbm.at[idx_v.at[j]],
                             rows_v.at[pl.ds(j * _G, _G)], sem)
            for j in range(_NG)
        ]
        for cp in cps:
            cp.wait()

        def group(g, carry2):
            wbase = pl.multiple_of(g * _GE, _LANES)
            wregs = [wv[pl.ds(wbase + k * _LANES, _LANES)]
                     for k in range(_GE // _LANES)]
            rbase = g * _GE
            for j in range(_GB):
                a0 = jnp.zeros((_LANES,), jnp.float32)
                a1 = jnp.zeros((_LANES,), jnp.float32)
                for e in range(_L):
                    f = j * _L + e
                    w = wregs[f // _LANES][f % _LANES]
                    a0 = a0 + w * rows_v[rbase + f, pl.ds(0, _LANES)]
                    a1 = a1 + w * rows_v[rbase + f, pl.ds(_LANES, _LANES)]
                ob = pl.multiple_of((g * _GB + j) * _D, _D)
                out_v[pl.ds(ob, _LANES)] = a0
                out_v[pl.ds(ob + _LANES, _LANES)] = a1
            return carry2

        lax.fori_loop(0, _C // _GB, group, 0)
        pltpu.sync_copy(out_v, o_hbm.at[pl.ds(cid * _C * _D, _C * _D)])
        return carry

    lax.fori_loop(0, num_chunks, chunk, 0)


def kernel(hashes, weights, table):
    info = plsc.get_sparse_core_info()
    nw = info.num_cores * info.num_subcores
    num_chunks = _B // (nw * _C)

    h2 = hashes.reshape(_B * _L // _G, _G)
    w1 = weights.reshape(_B * _L)

    mesh = plsc.VectorSubcoreMesh(core_axis_name="c", subcore_axis_name="s")
    run = functools.partial(
        pl.kernel,
        mesh=mesh,
        compiler_params=pltpu.CompilerParams(use_tc_tiling_on_sc=False),
        out_type=jax.ShapeDtypeStruct((_B * _D,), jnp.float32),
        scratch_types=[
            pltpu.VMEM((_NG, _G), jnp.int32),
            pltpu.VMEM((_E,), jnp.float32),
            pltpu.VMEM((_E, _D), jnp.float32),
            pltpu.VMEM((_C * _D,), jnp.float32),
            pltpu.SemaphoreType.DMA,
        ],
    )(functools.partial(_bag_kernel, num_cores=info.num_cores,
                        num_chunks=num_chunks))
    out = run(h2, w1, table)
    return out.reshape(_B, _D)


# speedup vs baseline: 3.0206x; 1.0938x over previous
import functools

import jax
import jax.numpy as jnp
from jax import lax
from jax.experimental import pallas as pl
from jax.experimental.pallas import tpu as pltpu
from jax.experimental.pallas import tpu_sc as plsc

_B = 16384
_L = 50
_D = 32
_LANES = 16

_C = 64
_E = _C * _L
_G = 128
_NG = _E // _G
_GB = 8
_GE = _GB * _L


_TCB = 512


def _transpose_block(x_ref, o_ref):
    for c in range(4):
        o_ref[:, c * _D:(c + 1) * _D] = jnp.transpose(
            x_ref[:, c * _TCB:(c + 1) * _TCB])


def _bag_kernel(h_hbm, w_hbm, t_hbm, o_hbm, idx_v, wv, rows_v, out_v, sem,
                *, num_cores, num_chunks):
    wid = lax.axis_index("s") * num_cores + lax.axis_index("c")

    def chunk(ci, carry):
        cid = wid * num_chunks + ci
        pltpu.sync_copy(h_hbm.at[pl.ds(cid * _NG, _NG)], idx_v)
        pltpu.sync_copy(w_hbm.at[pl.ds(cid * _E, _E)], wv)
        cps = [
            pltpu.async_copy(t_hbm.at[idx_v.at[j]],
                             rows_v.at[pl.ds(j * _G, _G)], sem)
            for j in range(_NG)
        ]
        for cp in cps:
            cp.wait()

        def group(g, carry2):
            wbase = pl.multiple_of(g * _GE, _LANES)
            wregs = [wv[pl.ds(wbase + k * _LANES, _LANES)]
                     for k in range(_GE // _LANES)]
            rbase = g * _GE
            for j in range(_GB):
                a0 = jnp.zeros((_LANES,), jnp.float32)
                a1 = jnp.zeros((_LANES,), jnp.float32)
                for e in range(_L):
                    f = j * _L + e
                    w = wregs[f // _LANES][f % _LANES]
                    a0 = a0 + w * rows_v[rbase + f, pl.ds(0, _LANES)]
                    a1 = a1 + w * rows_v[rbase + f, pl.ds(_LANES, _LANES)]
                ob = pl.multiple_of((g * _GB + j) * _D, _D)
                out_v[pl.ds(ob, _LANES)] = a0
                out_v[pl.ds(ob + _LANES, _LANES)] = a1
            return carry2

        lax.fori_loop(0, _C // _GB, group, 0)
        pltpu.sync_copy(out_v, o_hbm.at[pl.ds(cid * _C * _D, _C * _D)])
        return carry

    lax.fori_loop(0, num_chunks, chunk, 0)


def kernel(hashes, weights, table):
    info = plsc.get_sparse_core_info()
    nw = info.num_cores * info.num_subcores
    num_chunks = _B // (nw * _C)

    j = hashes % (4 * _TCB)
    hsig = (hashes - j) + 4 * (j % _TCB) + (j // _TCB)
    h2 = hsig.reshape(_B * _L // _G, _G)
    w1 = weights.reshape(_B * _L)

    n_rows = table.shape[0]
    n_blocks = pl.cdiv(n_rows, 4 * _TCB)
    n_pad_rows = n_blocks * _TCB
    tp = pl.pallas_call(
        _transpose_block,
        grid=(n_blocks,),
        in_specs=[pl.BlockSpec((_D, 4 * _TCB), lambda i: (0, i))],
        out_specs=pl.BlockSpec((_TCB, 4 * _D), lambda i: (i, 0)),
        out_shape=jax.ShapeDtypeStruct((n_pad_rows, 4 * _D), jnp.float32),
    )(table.T)
    tbl_lin = tp.reshape(n_pad_rows * 4, _D)

    mesh = plsc.VectorSubcoreMesh(core_axis_name="c", subcore_axis_name="s")
    run = functools.partial(
        pl.kernel,
        mesh=mesh,
        compiler_params=pltpu.CompilerParams(use_tc_tiling_on_sc=False),
        out_type=jax.ShapeDtypeStruct((_B * _D,), jnp.float32),
        scratch_types=[
            pltpu.VMEM((_NG, _G), jnp.int32),
            pltpu.VMEM((_E,), jnp.float32),
            pltpu.VMEM((_E, _D), jnp.float32),
            pltpu.VMEM((_C * _D,), jnp.float32),
            pltpu.SemaphoreType.DMA,
        ],
    )(functools.partial(_bag_kernel, num_cores=info.num_cores,
                        num_chunks=num_chunks))
    out = run(h2, w1, tbl_lin)
    return out.reshape(_B, _D)


# speedup vs baseline: 3.7980x; 1.2574x over previous
import functools

import jax
import jax.numpy as jnp
from jax import lax
from jax.experimental import pallas as pl
from jax.experimental.pallas import tpu as pltpu
from jax.experimental.pallas import tpu_sc as plsc

_B = 16384
_L = 50
_D = 32
_LANES = 16

_C = 64
_E = _C * _L
_G = 128
_NG = _E // _G
_GB = 8
_GE = _GB * _L


_TCB = 1024


def _transpose_block(x_ref, o_ref):
    eye = (lax.broadcasted_iota(jnp.int32, (_D, _D), 0) ==
           lax.broadcasted_iota(jnp.int32, (_D, _D), 1)).astype(jnp.float32)
    parts = [
        lax.dot_general(x_ref[:, c * _TCB:(c + 1) * _TCB], eye,
                        (((0,), (0,)), ((), ())),
                        preferred_element_type=jnp.float32)
        for c in range(4)
    ]
    o_ref[...] = jnp.concatenate(parts, axis=1)


def _bag_kernel(h_hbm, w_hbm, t_hbm, o_hbm, idx_v, wv, rows_v, out_v, sem,
                *, num_cores, num_chunks):
    wid = lax.axis_index("s") * num_cores + lax.axis_index("c")

    def chunk(ci, carry):
        cid = wid * num_chunks + ci
        pltpu.sync_copy(h_hbm.at[pl.ds(cid * _NG, _NG)], idx_v)
        pltpu.sync_copy(w_hbm.at[pl.ds(cid * _E, _E)], wv)
        cps = [
            pltpu.async_copy(t_hbm.at[idx_v.at[j]],
                             rows_v.at[pl.ds(j * _G, _G)], sem)
            for j in range(_NG)
        ]
        for cp in cps:
            cp.wait()

        def group(g, carry2):
            wbase = pl.multiple_of(g * _GE, _LANES)
            wregs = [wv[pl.ds(wbase + k * _LANES, _LANES)]
                     for k in range(_GE // _LANES)]
            rbase = g * _GE
            for j in range(_GB):
                a0 = jnp.zeros((_LANES,), jnp.float32)
                a1 = jnp.zeros((_LANES,), jnp.float32)
                for e in range(_L):
                    f = j * _L + e
                    w = wregs[f // _LANES][f % _LANES]
                    a0 = a0 + w * rows_v[rbase + f, pl.ds(0, _LANES)]
                    a1 = a1 + w * rows_v[rbase + f, pl.ds(_LANES, _LANES)]
                ob = pl.multiple_of((g * _GB + j) * _D, _D)
                out_v[pl.ds(ob, _LANES)] = a0
                out_v[pl.ds(ob + _LANES, _LANES)] = a1
            return carry2

        lax.fori_loop(0, _C // _GB, group, 0)
        pltpu.sync_copy(out_v, o_hbm.at[pl.ds(cid * _C * _D, _C * _D)])
        return carry

    lax.fori_loop(0, num_chunks, chunk, 0)


def kernel(hashes, weights, table):
    info = plsc.get_sparse_core_info()
    nw = info.num_cores * info.num_subcores
    num_chunks = _B // (nw * _C)

    j = hashes % (4 * _TCB)
    hsig = (hashes - j) + 4 * (j % _TCB) + (j // _TCB)
    h2 = hsig.reshape(_B * _L // _G, _G)
    w1 = weights.reshape(_B * _L)

    n_rows = table.shape[0]
    n_blocks = pl.cdiv(n_rows, 4 * _TCB)
    n_pad_rows = n_blocks * _TCB
    tp = pl.pallas_call(
        _transpose_block,
        grid=(n_blocks,),
        in_specs=[pl.BlockSpec((_D, 4 * _TCB), lambda i: (0, i))],
        out_specs=pl.BlockSpec((_TCB, 4 * _D), lambda i: (i, 0)),
        out_shape=jax.ShapeDtypeStruct((n_pad_rows, 4 * _D), jnp.float32),
    )(table.T)
    tbl_lin = tp.reshape(n_pad_rows * 4, _D)

    mesh = plsc.VectorSubcoreMesh(core_axis_name="c", subcore_axis_name="s")
    run = functools.partial(
        pl.kernel,
        mesh=mesh,
        compiler_params=pltpu.CompilerParams(use_tc_tiling_on_sc=False),
        out_type=jax.ShapeDtypeStruct((_B * _D,), jnp.float32),
        scratch_types=[
            pltpu.VMEM((_NG, _G), jnp.int32),
            pltpu.VMEM((_E,), jnp.float32),
            pltpu.VMEM((_E, _D), jnp.float32),
            pltpu.VMEM((_C * _D,), jnp.float32),
            pltpu.SemaphoreType.DMA,
        ],
    )(functools.partial(_bag_kernel, num_cores=info.num_cores,
                        num_chunks=num_chunks))
    out = run(h2, w1, tbl_lin)
    return out.reshape(_B, _D)


# speedup vs baseline: 3.8272x; 1.0077x over previous
import functools

import jax
import jax.numpy as jnp
from jax import lax
from jax.experimental import pallas as pl
from jax.experimental.pallas import tpu as pltpu
from jax.experimental.pallas import tpu_sc as plsc

_B = 16384
_L = 50
_D = 32
_LANES = 16

_C = 64
_E = _C * _L
_G = 128
_NG = _E // _G
_GB = 8
_GE = _GB * _L


_TCB = 1024


def _transpose_block(x_ref, o_ref):
    parts = [jnp.transpose(x_ref[:, c * _TCB:(c + 1) * _TCB])
             for c in range(4)]
    o_ref[...] = jnp.concatenate(parts, axis=1)


def _bag_kernel(h_hbm, w_hbm, t_hbm, o_hbm, idx_v, wv, rows_v, out_v, sem,
                *, num_cores, num_chunks):
    wid = lax.axis_index("s") * num_cores + lax.axis_index("c")

    def chunk(ci, carry):
        cid = wid * num_chunks + ci
        pltpu.sync_copy(h_hbm.at[pl.ds(cid * _NG, _NG)], idx_v)
        pltpu.sync_copy(w_hbm.at[pl.ds(cid * _E, _E)], wv)
        cps = [
            pltpu.async_copy(t_hbm.at[idx_v.at[j]],
                             rows_v.at[pl.ds(j * _G, _G)], sem)
            for j in range(_NG)
        ]
        for cp in cps:
            cp.wait()

        def group(g, carry2):
            wbase = pl.multiple_of(g * _GE, _LANES)
            wregs = [wv[pl.ds(wbase + k * _LANES, _LANES)]
                     for k in range(_GE // _LANES)]
            rbase = g * _GE
            for j in range(_GB):
                a0 = jnp.zeros((_LANES,), jnp.float32)
                a1 = jnp.zeros((_LANES,), jnp.float32)
                for e in range(_L):
                    f = j * _L + e
                    w = wregs[f // _LANES][f % _LANES]
                    a0 = a0 + w * rows_v[rbase + f, pl.ds(0, _LANES)]
                    a1 = a1 + w * rows_v[rbase + f, pl.ds(_LANES, _LANES)]
                ob = pl.multiple_of((g * _GB + j) * _D, _D)
                out_v[pl.ds(ob, _LANES)] = a0
                out_v[pl.ds(ob + _LANES, _LANES)] = a1
            return carry2

        lax.fori_loop(0, _C // _GB, group, 0)
        pltpu.sync_copy(out_v, o_hbm.at[pl.ds(cid * _C * _D, _C * _D)])
        return carry

    lax.fori_loop(0, num_chunks, chunk, 0)


def kernel(hashes, weights, table):
    info = plsc.get_sparse_core_info()
    nw = info.num_cores * info.num_subcores
    num_chunks = _B // (nw * _C)

    j = hashes % (4 * _TCB)
    hsig = (hashes - j) + 4 * (j % _TCB) + (j // _TCB)
    h2 = hsig.reshape(_B * _L // _G, _G)
    w1 = weights.reshape(_B * _L)

    n_rows = table.shape[0]
    n_blocks = pl.cdiv(n_rows, 4 * _TCB)
    n_pad_rows = n_blocks * _TCB
    tp = pl.pallas_call(
        _transpose_block,
        grid=(n_blocks,),
        in_specs=[pl.BlockSpec((_D, 4 * _TCB), lambda i: (0, i))],
        out_specs=pl.BlockSpec((_TCB, 4 * _D), lambda i: (i, 0)),
        out_shape=jax.ShapeDtypeStruct((n_pad_rows, 4 * _D), jnp.float32),
    )(table.T)
    tbl_lin = tp.reshape(n_pad_rows * 4, _D)

    mesh = plsc.VectorSubcoreMesh(core_axis_name="c", subcore_axis_name="s")
    run = functools.partial(
        pl.kernel,
        mesh=mesh,
        compiler_params=pltpu.CompilerParams(use_tc_tiling_on_sc=False),
        out_type=jax.ShapeDtypeStruct((_B * _D,), jnp.float32),
        scratch_types=[
            pltpu.VMEM((_NG, _G), jnp.int32),
            pltpu.VMEM((_E,), jnp.float32),
            pltpu.VMEM((_E, _D), jnp.float32),
            pltpu.VMEM((_C * _D,), jnp.float32),
            pltpu.SemaphoreType.DMA,
        ],
    )(functools.partial(_bag_kernel, num_cores=info.num_cores,
                        num_chunks=num_chunks))
    out = run(h2, w1, tbl_lin)
    return out.reshape(_B, _D)


# speedup vs baseline: 4.2337x; 1.1062x over previous
import functools

import jax
import jax.numpy as jnp
from jax import lax
from jax.experimental import pallas as pl
from jax.experimental.pallas import tpu as pltpu
from jax.experimental.pallas import tpu_sc as plsc

_B = 16384
_L = 50
_D = 32
_LANES = 16

_C = 64
_E = _C * _L
_G = 128
_NG = _E // _G
_GB = 8
_GE = _GB * _L


_TCB = 2048


def _transpose_block(x_ref, o_ref):
    parts = [jnp.transpose(x_ref[:, c * _TCB:(c + 1) * _TCB])
             for c in range(4)]
    o_ref[...] = jnp.concatenate(parts, axis=1)


def _bag_kernel(h_hbm, w_hbm, t_hbm, o_hbm, idx_v, wv, rows_v, out_v, sem,
                *, num_cores, num_chunks):
    wid = lax.axis_index("s") * num_cores + lax.axis_index("c")

    def chunk(ci, carry):
        cid = wid * num_chunks + ci
        pltpu.sync_copy(h_hbm.at[pl.ds(cid * _NG, _NG)], idx_v)
        pltpu.sync_copy(w_hbm.at[pl.ds(cid * _E, _E)], wv)
        cps = [
            pltpu.async_copy(t_hbm.at[idx_v.at[j]],
                             rows_v.at[pl.ds(j * _G, _G)], sem)
            for j in range(_NG)
        ]
        for cp in cps:
            cp.wait()

        def group(g, carry2):
            wbase = pl.multiple_of(g * _GE, _LANES)
            wregs = [wv[pl.ds(wbase + k * _LANES, _LANES)]
                     for k in range(_GE // _LANES)]
            rbase = g * _GE
            for j in range(_GB):
                a0 = jnp.zeros((_LANES,), jnp.float32)
                a1 = jnp.zeros((_LANES,), jnp.float32)
                for e in range(_L):
                    f = j * _L + e
                    w = wregs[f // _LANES][f % _LANES]
                    a0 = a0 + w * rows_v[rbase + f, pl.ds(0, _LANES)]
                    a1 = a1 + w * rows_v[rbase + f, pl.ds(_LANES, _LANES)]
                ob = pl.multiple_of((g * _GB + j) * _D, _D)
                out_v[pl.ds(ob, _LANES)] = a0
                out_v[pl.ds(ob + _LANES, _LANES)] = a1
            return carry2

        lax.fori_loop(0, _C // _GB, group, 0)
        pltpu.sync_copy(out_v, o_hbm.at[pl.ds(cid * _C * _D, _C * _D)])
        return carry

    lax.fori_loop(0, num_chunks, chunk, 0)


def kernel(hashes, weights, table):
    info = plsc.get_sparse_core_info()
    nw = info.num_cores * info.num_subcores
    num_chunks = _B // (nw * _C)

    j = hashes % (4 * _TCB)
    hsig = (hashes - j) + 4 * (j % _TCB) + (j // _TCB)
    h2 = hsig.reshape(_B * _L // _G, _G)
    w1 = weights.reshape(_B * _L)

    n_rows = table.shape[0]
    n_blocks = pl.cdiv(n_rows, 4 * _TCB)
    n_pad_rows = n_blocks * _TCB
    tp = pl.pallas_call(
        _transpose_block,
        grid=(n_blocks,),
        in_specs=[pl.BlockSpec((_D, 4 * _TCB), lambda i: (0, i))],
        out_specs=pl.BlockSpec((_TCB, 4 * _D), lambda i: (i, 0)),
        out_shape=jax.ShapeDtypeStruct((n_pad_rows, 4 * _D), jnp.float32),
    )(table.T)
    tbl_lin = tp.reshape(n_pad_rows * 4, _D)

    mesh = plsc.VectorSubcoreMesh(core_axis_name="c", subcore_axis_name="s")
    run = functools.partial(
        pl.kernel,
        mesh=mesh,
        compiler_params=pltpu.CompilerParams(use_tc_tiling_on_sc=False),
        out_type=jax.ShapeDtypeStruct((_B * _D,), jnp.float32),
        scratch_types=[
            pltpu.VMEM((_NG, _G), jnp.int32),
            pltpu.VMEM((_E,), jnp.float32),
            pltpu.VMEM((_E, _D), jnp.float32),
            pltpu.VMEM((_C * _D,), jnp.float32),
            pltpu.SemaphoreType.DMA,
        ],
    )(functools.partial(_bag_kernel, num_cores=info.num_cores,
                        num_chunks=num_chunks))
    out = run(h2, w1, tbl_lin)
    return out.reshape(_B, _D)


# speedup vs baseline: 4.2754x; 1.0098x over previous
import functools

import jax
import jax.numpy as jnp
from jax import lax
from jax.experimental import pallas as pl
from jax.experimental.pallas import tpu as pltpu
from jax.experimental.pallas import tpu_sc as plsc

_B = 16384
_L = 50
_D = 32
_LANES = 16

_C = 64
_E = _C * _L
_G = 128
_NG = _E // _G
_GB = 8
_GE = _GB * _L


_TCB = 4096


def _transpose_block(x_ref, o_ref):
    parts = [jnp.transpose(x_ref[:, c * _TCB:(c + 1) * _TCB])
             for c in range(4)]
    o_ref[...] = jnp.concatenate(parts, axis=1)


def _bag_kernel(h_hbm, w_hbm, t_hbm, o_hbm, idx_v, wv, rows_v, out_v, sem,
                *, num_cores, num_chunks):
    wid = lax.axis_index("s") * num_cores + lax.axis_index("c")

    def chunk(ci, carry):
        cid = wid * num_chunks + ci
        pltpu.sync_copy(h_hbm.at[pl.ds(cid * _NG, _NG)], idx_v)
        pltpu.sync_copy(w_hbm.at[pl.ds(cid * _E, _E)], wv)
        cps = [
            pltpu.async_copy(t_hbm.at[idx_v.at[j]],
                             rows_v.at[pl.ds(j * _G, _G)], sem)
            for j in range(_NG)
        ]
        for cp in cps:
            cp.wait()

        def group(g, carry2):
            wbase = pl.multiple_of(g * _GE, _LANES)
            wregs = [wv[pl.ds(wbase + k * _LANES, _LANES)]
                     for k in range(_GE // _LANES)]
            rbase = g * _GE
            for j in range(_GB):
                a0 = jnp.zeros((_LANES,), jnp.float32)
                a1 = jnp.zeros((_LANES,), jnp.float32)
                for e in range(_L):
                    f = j * _L + e
                    w = wregs[f // _LANES][f % _LANES]
                    a0 = a0 + w * rows_v[rbase + f, pl.ds(0, _LANES)]
                    a1 = a1 + w * rows_v[rbase + f, pl.ds(_LANES, _LANES)]
                ob = pl.multiple_of((g * _GB + j) * _D, _D)
                out_v[pl.ds(ob, _LANES)] = a0
                out_v[pl.ds(ob + _LANES, _LANES)] = a1
            return carry2

        lax.fori_loop(0, _C // _GB, group, 0)
        pltpu.sync_copy(out_v, o_hbm.at[pl.ds(cid * _C * _D, _C * _D)])
        return carry

    lax.fori_loop(0, num_chunks, chunk, 0)


def kernel(hashes, weights, table):
    info = plsc.get_sparse_core_info()
    nw = info.num_cores * info.num_subcores
    num_chunks = _B // (nw * _C)

    j = hashes % (4 * _TCB)
    hsig = (hashes - j) + 4 * (j % _TCB) + (j // _TCB)
    h2 = hsig.reshape(_B * _L // _G, _G)
    w1 = weights.reshape(_B * _L)

    n_rows = table.shape[0]
    n_blocks = pl.cdiv(n_rows, 4 * _TCB)
    n_pad_rows = n_blocks * _TCB
    tp = pl.pallas_call(
        _transpose_block,
        grid=(n_blocks,),
        in_specs=[pl.BlockSpec((_D, 4 * _TCB), lambda i: (0, i))],
        out_specs=pl.BlockSpec((_TCB, 4 * _D), lambda i: (i, 0)),
        out_shape=jax.ShapeDtypeStruct((n_pad_rows, 4 * _D), jnp.float32),
    )(table.T)
    tbl_lin = tp.reshape(n_pad_rows * 4, _D)

    mesh = plsc.VectorSubcoreMesh(core_axis_name="c", subcore_axis_name="s")
    run = functools.partial(
        pl.kernel,
        mesh=mesh,
        compiler_params=pltpu.CompilerParams(use_tc_tiling_on_sc=False),
        out_type=jax.ShapeDtypeStruct((_B * _D,), jnp.float32),
        scratch_types=[
            pltpu.VMEM((_NG, _G), jnp.int32),
            pltpu.VMEM((_E,), jnp.float32),
            pltpu.VMEM((_E, _D), jnp.float32),
            pltpu.VMEM((_C * _D,), jnp.float32),
            pltpu.SemaphoreType.DMA,
        ],
    )(functools.partial(_bag_kernel, num_cores=info.num_cores,
                        num_chunks=num_chunks))
    out = run(h2, w1, tbl_lin)
    return out.reshape(_B, _D)


# speedup vs baseline: 4.4798x; 1.0478x over previous
import functools

import jax
import jax.numpy as jnp
from jax import lax
from jax.experimental import pallas as pl
from jax.experimental.pallas import tpu as pltpu
from jax.experimental.pallas import tpu_sc as plsc

_B = 16384
_L = 50
_D = 32
_LANES = 16

_C = 32
_E = _C * _L
_G = 80
_NG = _E // _G
_GB = 8
_GE = _GB * _L


_TCB = 4096


def _transpose_block(x_ref, o_ref):
    parts = [jnp.transpose(x_ref[:, c * _TCB:(c + 1) * _TCB])
             for c in range(4)]
    o_ref[...] = jnp.concatenate(parts, axis=1)


def _bag_kernel(h_hbm, w_hbm, t_hbm, o_hbm,
                idx0, idx1, wv0, wv1, rows0, rows1, out_v, sem0, sem1,
                *, num_cores, num_chunks):
    wid = lax.axis_index("s") * num_cores + lax.axis_index("c")
    base = wid * num_chunks

    def fire(ci, idx_v, wv, rows_v, sem):
        cid = base + ci
        pltpu.sync_copy(h_hbm.at[pl.ds(cid * _E, _E)], idx_v)
        pltpu.sync_copy(w_hbm.at[pl.ds(cid * _E, _E)], wv)
        for j in range(_NG):
            pltpu.async_copy(t_hbm.at[idx_v.at[pl.ds(j * _G, _G)]],
                             rows_v.at[pl.ds(j * _G, _G)], sem)

    def drain(rows_v, sem):
        pltpu.make_async_copy(t_hbm.at[pl.ds(0, _E)], rows_v, sem).wait()

    def compute(ci, wv, rows_v):
        cid = base + ci

        def group(g, carry2):
            wbase = pl.multiple_of(g * _GE, _LANES)
            wregs = [wv[pl.ds(wbase + k * _LANES, _LANES)]
                     for k in range(_GE // _LANES)]
            rbase = g * _GE
            for j in range(_GB):
                a0 = jnp.zeros((_LANES,), jnp.float32)
                a1 = jnp.zeros((_LANES,), jnp.float32)
                for e in range(_L):
                    f = j * _L + e
                    w = wregs[f // _LANES][f % _LANES]
                    a0 = a0 + w * rows_v[rbase + f, pl.ds(0, _LANES)]
                    a1 = a1 + w * rows_v[rbase + f, pl.ds(_LANES, _LANES)]
                ob = pl.multiple_of((g * _GB + j) * _D, _D)
                out_v[pl.ds(ob, _LANES)] = a0
                out_v[pl.ds(ob + _LANES, _LANES)] = a1
            return carry2

        lax.fori_loop(0, _C // _GB, group, 0)
        pltpu.sync_copy(out_v, o_hbm.at[pl.ds(cid * _C * _D, _C * _D)])

    fire(0, idx0, wv0, rows0, sem0)

    def pair(i2, carry):
        c0 = i2 * 2
        fire(c0 + 1, idx1, wv1, rows1, sem1)
        drain(rows0, sem0)
        compute(c0, wv0, rows0)

        @pl.when(i2 + 1 < num_chunks // 2)
        def _():
            fire(c0 + 2, idx0, wv0, rows0, sem0)

        drain(rows1, sem1)
        compute(c0 + 1, wv1, rows1)
        return carry

    lax.fori_loop(0, num_chunks // 2, pair, 0)


def kernel(hashes, weights, table):
    info = plsc.get_sparse_core_info()
    nw = info.num_cores * info.num_subcores
    num_chunks = _B // (nw * _C)

    j = hashes % (4 * _TCB)
    hsig = (hashes - j) + 4 * (j % _TCB) + (j // _TCB)
    h1 = hsig.reshape(_B * _L)
    w1 = weights.reshape(_B * _L)

    n_rows = table.shape[0]
    n_blocks = pl.cdiv(n_rows, 4 * _TCB)
    n_pad_rows = n_blocks * _TCB
    tp = pl.pallas_call(
        _transpose_block,
        grid=(n_blocks,),
        in_specs=[pl.BlockSpec((_D, 4 * _TCB), lambda i: (0, i))],
        out_specs=pl.BlockSpec((_TCB, 4 * _D), lambda i: (i, 0)),
        out_shape=jax.ShapeDtypeStruct((n_pad_rows, 4 * _D), jnp.float32),
    )(table.T)
    tbl_lin = tp.reshape(n_pad_rows * 4, _D)

    mesh = plsc.VectorSubcoreMesh(core_axis_name="c", subcore_axis_name="s")
    run = functools.partial(
        pl.kernel,
        mesh=mesh,
        compiler_params=pltpu.CompilerParams(use_tc_tiling_on_sc=False),
        out_type=jax.ShapeDtypeStruct((_B * _D,), jnp.float32),
        scratch_types=[
            pltpu.VMEM((_E,), jnp.int32),
            pltpu.VMEM((_E,), jnp.int32),
            pltpu.VMEM((_E,), jnp.float32),
            pltpu.VMEM((_E,), jnp.float32),
            pltpu.VMEM((_E, _D), jnp.float32),
            pltpu.VMEM((_E, _D), jnp.float32),
            pltpu.VMEM((_C * _D,), jnp.float32),
            pltpu.SemaphoreType.DMA,
            pltpu.SemaphoreType.DMA,
        ],
    )(functools.partial(_bag_kernel, num_cores=info.num_cores,
                        num_chunks=num_chunks))
    out = run(h1, w1, tbl_lin)
    return out.reshape(_B, _D)
